# Initial kernel scaffold; baseline (speedup 1.0000x reference)
import jax
import jax.numpy as jnp
from jax.experimental import pallas as pl


def kernel(x, enc_w1, enc_w2, enc_w3, quant_w, codebook, post_w, dec_w1, dec_w2, dec_w3, head_onset_w, head_dur_w):
    raise NotImplementedError("write your pallas kernel here")


# speedup vs baseline: 1.8841x; 1.8841x over previous
import functools

import jax
import jax.numpy as jnp
from jax.experimental import pallas as pl

F32 = jnp.float32


def _conv_body(nr, nc, wout, cin, cout, relu, *refs):
    xs = refs[:nr]
    w_ref = refs[nr]
    out_ref = refs[nr + 1]
    th = out_ref.shape[1]
    acc = jnp.zeros((th * wout, cout), F32)
    for a in range(nr):
        xa = xs[a][0]
        for b in range(nc):
            blk = xa[:, b:b + wout, :].reshape(th * wout, cin)
            acc = acc + jnp.dot(blk, w_ref[a, b], preferred_element_type=F32)
    if relu:
        acc = jnp.maximum(acc, 0.0)
    out_ref[0] = acc.reshape(th, wout, cout)


def _tap_conv(x_pad, w, nr, nc, th, relu):
    b, hp, wp, cin = x_pad.shape
    h = hp - (nr - 1)
    wout = wp - (nc - 1)
    cout = w.shape[-1]
    views = [x_pad[:, d:d + h] for d in range(nr)]
    grid = (b, h // th)
    in_specs = [
        pl.BlockSpec((1, th, wp, cin), lambda i, j: (i, j, 0, 0))
        for _ in range(nr)
    ] + [pl.BlockSpec(w.shape, lambda i, j: (0, 0, 0, 0))]
    body = functools.partial(_conv_body, nr, nc, wout, cin, cout, relu)
    return pl.pallas_call(
        body,
        grid=grid,
        in_specs=in_specs,
        out_specs=pl.BlockSpec((1, th, wout, cout), lambda i, j: (i, j, 0, 0)),
        out_shape=jax.ShapeDtypeStruct((b, h, wout, cout), F32),
    )(*views, w)


def _upconv_body(wout, cin, cout, *refs):
    xs = refs[:3]
    w_ref = refs[3]
    outs = refs[4:8]
    th = outs[0].shape[1]
    for p in range(2):
        for q in range(2):
            acc = jnp.zeros((th * wout, cout), F32)
            for a in range(2):
                xa = xs[a + p][0]
                for b in range(2):
                    blk = xa[:, b + q:b + q + wout, :].reshape(th * wout, cin)
                    acc = acc + jnp.dot(blk, w_ref[p, q, a, b],
                                        preferred_element_type=F32)
            acc = jnp.maximum(acc, 0.0)
            outs[p * 2 + q][0] = acc.reshape(th, wout, cout)


def _upsample_conv(x, w3, th):
    b, h, wd, cin = x.shape
    cout = w3.shape[-1]
    rows = [
        jnp.stack([w3[0], w3[1] + w3[2]]),
        jnp.stack([w3[0] + w3[1], w3[2]]),
    ]
    wpq = []
    for p in range(2):
        t = rows[p]
        cols = [
            jnp.stack([t[:, 0], t[:, 1] + t[:, 2]], axis=1),
            jnp.stack([t[:, 0] + t[:, 1], t[:, 2]], axis=1),
        ]
        wpq.append(jnp.stack(cols))
    w = jnp.stack(wpq)

    x_pad = jnp.pad(x, ((0, 0), (1, 1), (1, 1), (0, 0)))
    views = [x_pad[:, d:d + h] for d in range(3)]
    grid = (b, h // th)
    in_specs = [
        pl.BlockSpec((1, th, wd + 2, cin), lambda i, j: (i, j, 0, 0))
        for _ in range(3)
    ] + [pl.BlockSpec(w.shape, lambda i, j: (0,) * 6)]
    out_specs = [
        pl.BlockSpec((1, th, wd, cout), lambda i, j: (i, j, 0, 0))
        for _ in range(4)
    ]
    out_shape = [jax.ShapeDtypeStruct((b, h, wd, cout), F32)] * 4
    body = functools.partial(_upconv_body, wd, cin, cout)
    y00, y01, y10, y11 = pl.pallas_call(
        body, grid=grid, in_specs=in_specs, out_specs=out_specs,
        out_shape=out_shape,
    )(*views, w)
    y0 = jnp.stack([y00, y01], axis=3)
    y1 = jnp.stack([y10, y11], axis=3)
    y = jnp.stack([y0, y1], axis=2)
    return y.reshape(b, 2 * h, 2 * wd, cout)


def _quant_body(zf_ref, cb_ref, q_ref, s_ref):
    z = zf_ref[...]
    cb = cb_ref[...]
    zsq = jnp.sum(z * z, axis=1, keepdims=True)
    csq = jnp.sum(cb * cb, axis=1)
    cross = jax.lax.dot_general(z, cb, (((1,), (1,)), ((), ())),
                                preferred_element_type=F32)
    d = zsq - 2.0 * cross + csq[None, :]
    iota = jax.lax.broadcasted_iota(jnp.int32, d.shape, 1)
    mind = jnp.min(d, axis=1, keepdims=True)
    idx = jnp.min(jnp.where(d == mind, iota, jnp.int32(1 << 30)), axis=1)
    onehot = (iota == idx[:, None]).astype(F32)
    q = jnp.dot(onehot, cb, preferred_element_type=F32)
    diff = q - z
    q_ref[...] = q
    s_ref[0, 0, :] = jnp.full((128,), jnp.sum(diff * diff), F32)


def _quantize(zf, codebook, blk):
    n, c = zf.shape
    k = codebook.shape[0]
    grid = (n // blk,)
    q, sums = pl.pallas_call(
        _quant_body,
        grid=grid,
        in_specs=[
            pl.BlockSpec((blk, c), lambda i: (i, 0)),
            pl.BlockSpec((k, c), lambda i: (0, 0)),
        ],
        out_specs=[
            pl.BlockSpec((blk, c), lambda i: (i, 0)),
            pl.BlockSpec((1, 1, 128), lambda i: (i, 0, 0)),
        ],
        out_shape=[
            jax.ShapeDtypeStruct((n, c), F32),
            jax.ShapeDtypeStruct((grid[0], 1, 128), F32),
        ],
    )(zf, codebook)
    commit = 1.25 * jnp.sum(sums[:, 0, 0]) / (n * c)
    return q, commit


def _space_to_depth2(x_pad):
    b, he, we, c = x_pad.shape
    y = x_pad.reshape(b, he // 2, 2, we // 2, 2, c)
    y = jnp.transpose(y, (0, 1, 3, 2, 4, 5))
    return y.reshape(b, he // 2, we // 2, 4 * c)


def _s2d_weights(w_oihw):
    w = jnp.transpose(w_oihw, (2, 3, 1, 0))
    kh, kw, ci, co = w.shape
    w = w.reshape(2, 2, 2, 2, ci, co)
    w = jnp.transpose(w, (0, 2, 1, 3, 4, 5))
    return w.reshape(2, 2, 4 * ci, co)


def kernel(x, enc_w1, enc_w2, enc_w3, quant_w, codebook, post_w,
           dec_w1, dec_w2, dec_w3, head_onset_w, head_dur_w):
    b = x.shape[0]

    xh = jnp.transpose(x, (0, 2, 3, 1))
    xp = jnp.pad(xh, ((0, 0), (1, 1), (1, 1), (0, 0)))
    s1 = _space_to_depth2(xp)
    h1 = _tap_conv(s1, _s2d_weights(enc_w1), 2, 2, th=16, relu=True)

    h1p = jnp.pad(h1, ((0, 0), (1, 1), (1, 1), (0, 0)))
    s2 = _space_to_depth2(h1p)
    h2 = _tap_conv(s2, _s2d_weights(enc_w2), 2, 2, th=16, relu=True)

    w3 = jnp.transpose(enc_w3, (2, 3, 1, 0))
    h2p = jnp.pad(h2, ((0, 0), (1, 1), (1, 1), (0, 0)))
    enc = _tap_conv(h2p, w3, 3, 3, th=16, relu=False)
    qm = quant_w[:, :, 0, 0].T[None, None]
    hq = _tap_conv(enc, qm, 1, 1, th=16, relu=False)

    c = hq.shape[-1]
    zf = hq.reshape(-1, c)
    q, commit = _quantize(zf, codebook, blk=1024)
    quant = q.reshape(hq.shape)

    pm = post_w[:, :, 0, 0].T[None, None]
    post = _tap_conv(quant, pm, 1, 1, th=16, relu=False)
    wd1 = jnp.transpose(dec_w1, (2, 3, 1, 0))
    qp = jnp.pad(post, ((0, 0), (1, 1), (1, 1), (0, 0)))
    d1 = _tap_conv(qp, wd1, 3, 3, th=16, relu=True)

    d2 = _upsample_conv(d1, jnp.transpose(dec_w2, (2, 3, 1, 0)), th=16)
    d3 = _upsample_conv(d2, jnp.transpose(dec_w3, (2, 3, 1, 0)), th=16)

    wh = jnp.concatenate([head_onset_w, head_dur_w], axis=0)
    wh = jnp.transpose(wh, (2, 3, 1, 0))
    wh = jnp.pad(wh, ((0, 0), (0, 0), (0, 0), (0, 4)))
    d3p = jnp.pad(d3, ((0, 0), (1, 1), (1, 1), (0, 0)))
    heads = _tap_conv(d3p, wh, 3, 3, th=16, relu=False)

    dec_onset = jnp.transpose(heads[..., 0:2], (0, 3, 1, 2))
    dec_duration = jnp.transpose(heads[..., 2:4], (0, 3, 1, 2))
    return (dec_onset, dec_duration, commit)
